# baseline (speedup 1.0000x reference)
import functools

import jax
import jax.numpy as jnp
from jax import lax
from jax.experimental import pallas as pl
from jax.experimental.pallas import tpu as pltpu
from jax.experimental.pallas import tpu_sc as plsc

VOCAB_EXCL_PAD = 5
HIDDEN_DIM = 32
OUTPUT_DIM = 25
MAX_COUNT = 16.0
SEQ = 200

NC, NS, LANES = 2, 16, 16
NW = NC * NS
CHUNK = 64
TOK_PER_FLUSH = 25
N_FLUSH = SEQ // TOK_PER_FLUSH

BR = 512


def _make_sc_hist(B):
    rows_per_w = B // NW
    n_chunks = rows_per_w // CHUNK

    def body(ids_hbm, out_hbm, buf0, buf1, outbuf, sem0, sem1):
        cid = lax.axis_index("c")
        sid = lax.axis_index("s")
        wid = sid * NC + cid
        row0 = wid * rows_per_w
        bufs = (buf0, buf1)
        sems = (sem0, sem1)
        iota = lax.iota(jnp.int32, LANES)
        one = jnp.full((LANES,), 1, jnp.int32)

        copies = [None, None]
        copies[0] = pltpu.async_copy(ids_hbm.at[pl.ds(row0, CHUNK)], buf0, sem0)
        for ci in range(n_chunks):
            if ci + 1 < n_chunks:
                copies[(ci + 1) % 2] = pltpu.async_copy(
                    ids_hbm.at[pl.ds(row0 + (ci + 1) * CHUNK, CHUNK)],
                    bufs[(ci + 1) % 2], sems[(ci + 1) % 2])
            copies[ci % 2].wait()
            buf = bufs[ci % 2]
            for g in range(CHUNK // LANES):
                row_idx = iota + g * LANES

                def kbody(k, wides, buf=buf, row_idx=row_idx):
                    acc = jnp.zeros((LANES,), jnp.int32)
                    base = k * TOK_PER_FLUSH
                    for t in range(TOK_PER_FLUSH):
                        col = jnp.full((LANES,), base + t, jnp.int32)
                        c = plsc.load_gather(buf, [row_idx, col])
                        acc = acc + (one << ((c << 2) + c))
                    return tuple(
                        wides[v] + ((acc >> (5 * (v + 1))) & 31)
                        for v in range(VOCAB_EXCL_PAD))

                wides = lax.fori_loop(
                    0, N_FLUSH, kbody,
                    tuple(jnp.zeros((LANES,), jnp.int32)
                          for _ in range(VOCAB_EXCL_PAD)))
                for v in range(VOCAB_EXCL_PAD):
                    val = wides[v].astype(jnp.float32) * (1.0 / MAX_COUNT)
                    plsc.store_scatter(
                        outbuf, [row_idx, jnp.full((LANES,), v, jnp.int32)], val)
                for v in range(VOCAB_EXCL_PAD, 8):
                    plsc.store_scatter(
                        outbuf, [row_idx, jnp.full((LANES,), v, jnp.int32)],
                        jnp.zeros((LANES,), jnp.float32))
            pltpu.sync_copy(outbuf, out_hbm.at[pl.ds(row0 + ci * CHUNK, CHUNK)])

    mesh = plsc.VectorSubcoreMesh(core_axis_name="c", subcore_axis_name="s")
    return pl.kernel(
        body,
        out_type=jax.ShapeDtypeStruct((B, 8), jnp.float32),
        mesh=mesh,
        compiler_params=pltpu.CompilerParams(
            use_tc_tiling_on_sc=False, needs_layout_passes=False),
        scratch_types=[
            pltpu.VMEM((CHUNK, SEQ), jnp.int32),
            pltpu.VMEM((CHUNK, SEQ), jnp.int32),
            pltpu.VMEM((CHUNK, 8), jnp.float32),
            pltpu.SemaphoreType.DMA,
            pltpu.SemaphoreType.DMA,
        ],
    )


def _mlp_body(cnt_ref, w1t_ref, b1_ref, w2t_ref, b2_ref, gb_ref, out_ref):
    c = cnt_ref[...]
    h = jnp.dot(c, w1t_ref[...], preferred_element_type=jnp.float32)
    h = jnp.maximum(h + b1_ref[0, :][None, :], 0.0)
    h2 = jnp.dot(h, w2t_ref[...], preferred_element_type=jnp.float32)
    h2 = h2 + b2_ref[0, :][None, :]
    mean = jnp.mean(h2, axis=1, keepdims=True)
    d = h2 - mean
    var = jnp.mean(d * d, axis=1, keepdims=True)
    hn = d * lax.rsqrt(var + 1e-5)
    hn = hn * gb_ref[0, :][None, :] + gb_ref[1, :][None, :]
    out_ref[...] = jnp.maximum(hn, 0.0)


@jax.jit
def kernel(padded_ids, W1, b1, W2, b2, gamma, beta):
    B = padded_ids.shape[0]
    ids = padded_ids.astype(jnp.int32)

    counts = _make_sc_hist(B)(ids)

    w1t8 = jnp.zeros((8, HIDDEN_DIM), jnp.float32).at[:VOCAB_EXCL_PAD].set(W1.T)
    w2t = W2.T
    b1r = b1.reshape(1, HIDDEN_DIM)
    b2r = b2.reshape(1, OUTPUT_DIM)
    gb = jnp.stack([gamma, beta], axis=0)

    out = pl.pallas_call(
        _mlp_body,
        grid=(B // BR,),
        in_specs=[
            pl.BlockSpec((BR, 8), lambda i: (i, 0)),
            pl.BlockSpec((8, HIDDEN_DIM), lambda i: (0, 0)),
            pl.BlockSpec((1, HIDDEN_DIM), lambda i: (0, 0)),
            pl.BlockSpec((HIDDEN_DIM, OUTPUT_DIM), lambda i: (0, 0)),
            pl.BlockSpec((1, OUTPUT_DIM), lambda i: (0, 0)),
            pl.BlockSpec((2, OUTPUT_DIM), lambda i: (0, 0)),
        ],
        out_specs=pl.BlockSpec((BR, OUTPUT_DIM), lambda i: (i, 0)),
        out_shape=jax.ShapeDtypeStruct((B, OUTPUT_DIM), jnp.float32),
    )(counts, w1t8, b1r, w2t, b2r, gb)
    return out


# speedup vs baseline: 1.1799x; 1.1799x over previous
import functools

import jax
import jax.numpy as jnp
from jax import lax
from jax.experimental import pallas as pl
from jax.experimental.pallas import tpu as pltpu
from jax.experimental.pallas import tpu_sc as plsc

VOCAB_EXCL_PAD = 5
HIDDEN_DIM = 32
OUTPUT_DIM = 25
MAX_COUNT = 16.0
SEQ = 200

NC, NS, LANES = 2, 16, 16
NW = NC * NS
CHUNK = 64
TOK_PER_FLUSH = 25
N_FLUSH = SEQ // TOK_PER_FLUSH

BR = 2048


def _make_sc_hist(B):
    rows_per_w = B // NW
    n_chunks = rows_per_w // CHUNK

    def body(ids_hbm, out_hbm, buf0, buf1, outbuf, sem0, sem1):
        cid = lax.axis_index("c")
        sid = lax.axis_index("s")
        wid = sid * NC + cid
        row0 = wid * rows_per_w
        bufs = (buf0, buf1)
        sems = (sem0, sem1)
        iota = lax.iota(jnp.int32, LANES)
        one = jnp.full((LANES,), 1, jnp.int32)

        copies = [None, None]
        copies[0] = pltpu.async_copy(
            ids_hbm.at[pl.ds(row0 * SEQ, CHUNK * SEQ)], buf0, sem0)
        for ci in range(n_chunks):
            if ci + 1 < n_chunks:
                copies[(ci + 1) % 2] = pltpu.async_copy(
                    ids_hbm.at[pl.ds((row0 + (ci + 1) * CHUNK) * SEQ,
                                     CHUNK * SEQ)],
                    bufs[(ci + 1) % 2], sems[(ci + 1) % 2])
            copies[ci % 2].wait()
            buf = bufs[ci % 2]
            for g in range(CHUNK // LANES):
                rowbase = (iota + g * LANES) * SEQ

                def kbody(k, wides, buf=buf, rowbase=rowbase):
                    acc = jnp.zeros((LANES,), jnp.int32)
                    base = k * TOK_PER_FLUSH
                    for t in range(TOK_PER_FLUSH):
                        col = jnp.full((LANES,), base + t, jnp.int32)
                        c = plsc.load_gather(buf, [rowbase + col])
                        acc = acc + (one << ((c << 2) + c))
                    return tuple(
                        wides[v] + ((acc >> (5 * (v + 1))) & 31)
                        for v in range(VOCAB_EXCL_PAD))

                wides = lax.fori_loop(
                    0, N_FLUSH, kbody,
                    tuple(jnp.zeros((LANES,), jnp.int32)
                          for _ in range(VOCAB_EXCL_PAD)))
                out_idx = (iota + g * LANES) * 8
                for v in range(VOCAB_EXCL_PAD):
                    val = wides[v].astype(jnp.float32) * (1.0 / MAX_COUNT)
                    plsc.store_scatter(
                        outbuf, [out_idx + jnp.full((LANES,), v, jnp.int32)],
                        val)
                for v in range(VOCAB_EXCL_PAD, 8):
                    plsc.store_scatter(
                        outbuf, [out_idx + jnp.full((LANES,), v, jnp.int32)],
                        jnp.zeros((LANES,), jnp.float32))
            pltpu.sync_copy(
                outbuf, out_hbm.at[pl.ds((row0 + ci * CHUNK) * 8, CHUNK * 8)])

    mesh = plsc.VectorSubcoreMesh(core_axis_name="c", subcore_axis_name="s")
    return pl.kernel(
        body,
        out_type=jax.ShapeDtypeStruct((B * 8,), jnp.float32),
        mesh=mesh,
        compiler_params=pltpu.CompilerParams(
            use_tc_tiling_on_sc=False, needs_layout_passes=False),
        scratch_types=[
            pltpu.VMEM((CHUNK * SEQ,), jnp.int32),
            pltpu.VMEM((CHUNK * SEQ,), jnp.int32),
            pltpu.VMEM((CHUNK * 8,), jnp.float32),
            pltpu.SemaphoreType.DMA,
            pltpu.SemaphoreType.DMA,
        ],
    )


def _mlp_body(cnt_ref, w1t_ref, b1_ref, w2t_ref, b2_ref, gb_ref, out_ref):
    c = cnt_ref[...]
    h = jnp.dot(c, w1t_ref[...], preferred_element_type=jnp.float32)
    h = jnp.maximum(h + b1_ref[0, :][None, :], 0.0)
    h2 = jnp.dot(h, w2t_ref[...], preferred_element_type=jnp.float32)
    h2 = h2 + b2_ref[0, :][None, :]
    mean = jnp.mean(h2, axis=1, keepdims=True)
    d = h2 - mean
    var = jnp.mean(d * d, axis=1, keepdims=True)
    hn = d * lax.rsqrt(var + 1e-5)
    hn = hn * gb_ref[0, :][None, :] + gb_ref[1, :][None, :]
    out_ref[...] = jnp.maximum(hn, 0.0)


@jax.jit
def kernel(padded_ids, W1, b1, W2, b2, gamma, beta):
    B = padded_ids.shape[0]
    ids = padded_ids.astype(jnp.int32).reshape(B * SEQ)

    counts = _make_sc_hist(B)(ids).reshape(B, 8)

    w1t8 = jnp.zeros((8, HIDDEN_DIM), jnp.float32).at[:VOCAB_EXCL_PAD].set(W1.T)
    w2t = W2.T
    b1r = b1.reshape(1, HIDDEN_DIM)
    b2r = b2.reshape(1, OUTPUT_DIM)
    gb = jnp.stack([gamma, beta], axis=0)

    out = pl.pallas_call(
        _mlp_body,
        grid=(B // BR,),
        in_specs=[
            pl.BlockSpec((BR, 8), lambda i: (i, 0)),
            pl.BlockSpec((8, HIDDEN_DIM), lambda i: (0, 0)),
            pl.BlockSpec((1, HIDDEN_DIM), lambda i: (0, 0)),
            pl.BlockSpec((HIDDEN_DIM, OUTPUT_DIM), lambda i: (0, 0)),
            pl.BlockSpec((1, OUTPUT_DIM), lambda i: (0, 0)),
            pl.BlockSpec((2, OUTPUT_DIM), lambda i: (0, 0)),
        ],
        out_specs=pl.BlockSpec((BR, OUTPUT_DIM), lambda i: (i, 0)),
        out_shape=jax.ShapeDtypeStruct((B, OUTPUT_DIM), jnp.float32),
    )(counts, w1t8, b1r, w2t, b2r, gb)
    return out


# speedup vs baseline: 6.4055x; 5.4291x over previous
import functools

import jax
import jax.numpy as jnp
from jax import lax
from jax.experimental import pallas as pl
from jax.experimental.pallas import tpu as pltpu

VOCAB_EXCL_PAD = 5
HIDDEN_DIM = 32
OUTPUT_DIM = 25
MAX_COUNT = 16.0
SEQ = 200

BC = 2048
GROUP = 24


def _body(ids_ref, w1_ref, b1_ref, w2_ref, b2_ref, g_ref, bt_ref, out_ref):
    ids = ids_ref[...]
    packed = jnp.full(ids.shape, 1, jnp.int32) << ((ids << 2) + ids)
    wides = [jnp.zeros((1, ids.shape[1]), jnp.int32)
             for _ in range(VOCAB_EXCL_PAD)]
    for g0 in range(0, SEQ, GROUP):
        g1 = min(g0 + GROUP, SEQ)
        s = jnp.sum(packed[g0:g1], axis=0, keepdims=True)
        for v in range(VOCAB_EXCL_PAD):
            wides[v] = wides[v] + ((s >> (5 * (v + 1))) & 31)

    h = jnp.broadcast_to(b1_ref[...], (HIDDEN_DIM, ids.shape[1]))
    for v in range(VOCAB_EXCL_PAD):
        cnt = wides[v].astype(jnp.float32) * (1.0 / MAX_COUNT)
        h = h + w1_ref[:, v][:, None] * cnt
    h = jnp.maximum(h, 0.0)

    h2 = jnp.dot(w2_ref[...], h, preferred_element_type=jnp.float32)
    h2 = h2 + b2_ref[...]
    mean = jnp.mean(h2, axis=0, keepdims=True)
    d = h2 - mean
    var = jnp.mean(d * d, axis=0, keepdims=True)
    hn = d * lax.rsqrt(var + 1e-5)
    hn = hn * g_ref[...] + bt_ref[...]
    out_ref[...] = jnp.maximum(hn, 0.0)


@jax.jit
def kernel(padded_ids, W1, b1, W2, b2, gamma, beta):
    B = padded_ids.shape[0]
    ids_t = padded_ids.astype(jnp.int32).T

    out_t = pl.pallas_call(
        _body,
        grid=(B // BC,),
        in_specs=[
            pl.BlockSpec((SEQ, BC), lambda i: (0, i)),
            pl.BlockSpec((HIDDEN_DIM, VOCAB_EXCL_PAD), lambda i: (0, 0)),
            pl.BlockSpec((HIDDEN_DIM, 1), lambda i: (0, 0)),
            pl.BlockSpec((OUTPUT_DIM, HIDDEN_DIM), lambda i: (0, 0)),
            pl.BlockSpec((OUTPUT_DIM, 1), lambda i: (0, 0)),
            pl.BlockSpec((OUTPUT_DIM, 1), lambda i: (0, 0)),
            pl.BlockSpec((OUTPUT_DIM, 1), lambda i: (0, 0)),
        ],
        out_specs=pl.BlockSpec((OUTPUT_DIM, BC), lambda i: (0, i)),
        out_shape=jax.ShapeDtypeStruct((OUTPUT_DIM, B), jnp.float32),
    )(ids_t, W1, b1.reshape(HIDDEN_DIM, 1), W2, b2.reshape(OUTPUT_DIM, 1),
      gamma.reshape(OUTPUT_DIM, 1), beta.reshape(OUTPUT_DIM, 1))
    return out_t.T


# speedup vs baseline: 8.1062x; 1.2655x over previous
import functools

import jax
import jax.numpy as jnp
from jax import lax
from jax.experimental import pallas as pl
from jax.experimental.pallas import tpu as pltpu

VOCAB_EXCL_PAD = 5
HIDDEN_DIM = 32
OUTPUT_DIM = 25
MAX_COUNT = 16.0
SEQ = 200

BC = 2048
GROUP = 24


def _body(ids_ref, w1t_ref, w2_ref, p_ref, out_ref):
    ids = ids_ref[...]
    packed = jnp.full(ids.shape, 1, jnp.int32) << ((ids << 2) + ids)
    wides = [jnp.zeros((1, ids.shape[1]), jnp.int32)
             for _ in range(VOCAB_EXCL_PAD)]
    for g0 in range(0, SEQ, GROUP):
        g1 = min(g0 + GROUP, SEQ)
        s = jnp.sum(packed[g0:g1], axis=0, keepdims=True)
        for v in range(VOCAB_EXCL_PAD):
            wides[v] = wides[v] + ((s >> (5 * (v + 1))) & 31)
    counts = jnp.concatenate(wides, axis=0).astype(jnp.float32)
    counts = counts * (1.0 / MAX_COUNT)

    h = lax.dot_general(w1t_ref[...], counts, (((0,), (0,)), ((), ())),
                        preferred_element_type=jnp.float32)
    h = jnp.maximum(h + p_ref[:, 0][:, None], 0.0)

    h2 = jnp.dot(w2_ref[...], h, preferred_element_type=jnp.float32)
    h2 = h2 + p_ref[:OUTPUT_DIM, 1][:, None]
    mean = jnp.mean(h2, axis=0, keepdims=True)
    d = h2 - mean
    var = jnp.mean(d * d, axis=0, keepdims=True)
    hn = d * lax.rsqrt(var + 1e-5)
    hn = hn * p_ref[:OUTPUT_DIM, 2][:, None] + p_ref[:OUTPUT_DIM, 3][:, None]
    out_ref[...] = jnp.maximum(hn, 0.0)


@jax.jit
def kernel(padded_ids, W1, b1, W2, b2, gamma, beta):
    B = padded_ids.shape[0]
    ids_t = padded_ids.astype(jnp.int32).T
    w1t = W1.T
    pad = jnp.zeros((HIDDEN_DIM - OUTPUT_DIM,), jnp.float32)
    params = jnp.stack(
        [b1,
         jnp.concatenate([b2, pad]),
         jnp.concatenate([gamma, pad]),
         jnp.concatenate([beta, pad])], axis=1)

    out_t = pl.pallas_call(
        _body,
        grid=(B // BC,),
        in_specs=[
            pl.BlockSpec((SEQ, BC), lambda i: (0, i)),
            pl.BlockSpec((VOCAB_EXCL_PAD, HIDDEN_DIM), lambda i: (0, 0)),
            pl.BlockSpec((OUTPUT_DIM, HIDDEN_DIM), lambda i: (0, 0)),
            pl.BlockSpec((HIDDEN_DIM, 4), lambda i: (0, 0)),
        ],
        out_specs=pl.BlockSpec((OUTPUT_DIM, BC), lambda i: (0, i)),
        out_shape=jax.ShapeDtypeStruct((OUTPUT_DIM, B), jnp.float32),
    )(ids_t, w1t, W2, params)
    return out_t.T


# speedup vs baseline: 11.7051x; 1.4440x over previous
import functools

import jax
import jax.numpy as jnp
from jax import lax
from jax.experimental import pallas as pl
from jax.experimental.pallas import tpu as pltpu

VOCAB_EXCL_PAD = 5
HIDDEN_DIM = 32
OUTPUT_DIM = 25
MAX_COUNT = 16.0
SEQ = 200

BC = 4096
GROUP = 25


def _body(ids_ref, w1t_ref, w2_ref, b1_ref, b2_ref, g_ref, bt_ref, out_ref):
    b1c = jnp.transpose(b1_ref[...][None, :])
    b2c = jnp.transpose(b2_ref[...][None, :])
    gc = jnp.transpose(g_ref[...][None, :])
    btc = jnp.transpose(bt_ref[...][None, :])

    ids = ids_ref[...]
    packed = jnp.full(ids.shape, 1, jnp.int32) << ((ids << 2) + ids)
    wides = [jnp.zeros((1, ids.shape[1]), jnp.int32)
             for _ in range(VOCAB_EXCL_PAD)]
    for g0 in range(0, SEQ, GROUP):
        g1 = min(g0 + GROUP, SEQ)
        s = jnp.sum(packed[g0:g1], axis=0, keepdims=True)
        for v in range(VOCAB_EXCL_PAD):
            wides[v] = wides[v] + ((s >> (5 * (v + 1))) & 31)
    counts = jnp.concatenate(wides, axis=0).astype(jnp.float32)
    counts = counts * (1.0 / MAX_COUNT)

    h = lax.dot_general(w1t_ref[...], counts, (((0,), (0,)), ((), ())),
                        preferred_element_type=jnp.float32)
    h = jnp.maximum(h + b1c, 0.0)

    h2 = jnp.dot(w2_ref[...], h, preferred_element_type=jnp.float32)
    h2 = h2 + b2c
    mean = jnp.mean(h2, axis=0, keepdims=True)
    d = h2 - mean
    var = jnp.mean(d * d, axis=0, keepdims=True)
    hn = d * lax.rsqrt(var + 1e-5)
    hn = hn * gc + btc
    out_ref[...] = jnp.maximum(hn, 0.0)


@jax.jit
def kernel(padded_ids, W1, b1, W2, b2, gamma, beta):
    B = padded_ids.shape[0]
    ids_t = padded_ids.astype(jnp.int32).T
    w1t = W1.T

    out_t = pl.pallas_call(
        _body,
        grid=(B // BC,),
        in_specs=[
            pl.BlockSpec((SEQ, BC), lambda i: (0, i)),
            pl.BlockSpec((VOCAB_EXCL_PAD, HIDDEN_DIM), lambda i: (0, 0)),
            pl.BlockSpec((OUTPUT_DIM, HIDDEN_DIM), lambda i: (0, 0)),
            pl.BlockSpec((HIDDEN_DIM,), lambda i: (0,)),
            pl.BlockSpec((OUTPUT_DIM,), lambda i: (0,)),
            pl.BlockSpec((OUTPUT_DIM,), lambda i: (0,)),
            pl.BlockSpec((OUTPUT_DIM,), lambda i: (0,)),
        ],
        out_specs=pl.BlockSpec((OUTPUT_DIM, BC), lambda i: (0, i)),
        out_shape=jax.ShapeDtypeStruct((OUTPUT_DIM, B), jnp.float32),
    )(ids_t, w1t, W2, b1, b2, gamma, beta)
    return out_t.T
